# baseline (speedup 1.0000x reference)
import functools

import jax
import jax.numpy as jnp
from jax import lax
from jax.experimental import pallas as pl
from jax.experimental.pallas import tpu as pltpu
from jax.experimental.pallas import tpu_sc as plsc

N = 10000
DEG = 32
D = 128
NF = 4
VOCAB = 1000
G = 16
NCLS = 33

_SC_NC = 2
_SC_NS = 16
_SC_NW = _SC_NC * _SC_NS


@functools.lru_cache(maxsize=None)
def _make_sc_gather(V, B, C, W=D):
    del V
    bpw = B // _SC_NW
    assert B % _SC_NW == 0 and bpw % C == 0 and C % 8 == 0
    nch = bpw // C
    assert nch % 2 == 0
    mesh = plsc.VectorSubcoreMesh(core_axis_name="c", subcore_axis_name="s")

    @functools.partial(
        pl.kernel,
        mesh=mesh,
        compiler_params=pltpu.CompilerParams(use_tc_tiling_on_sc=False),
        out_type=jax.ShapeDtypeStruct((B, W), jnp.float32),
        scratch_types=[
            pltpu.VMEM((bpw,), jnp.int32),
            pltpu.VMEM((C, W), jnp.float32),
            pltpu.VMEM((C, W), jnp.float32),
            pltpu.SemaphoreType.DMA,
            pltpu.SemaphoreType.DMA,
            pltpu.SemaphoreType.DMA,
            pltpu.SemaphoreType.DMA,
        ],
    )
    def gather_kernel(table_hbm, idx_hbm, out_hbm, idx_v, buf0, buf1,
                      gs0, gs1, ws0, ws1):
        wid = lax.axis_index("s") * _SC_NC + lax.axis_index("c")
        base = wid * bpw
        pltpu.sync_copy(idx_hbm.at[pl.ds(base, bpw)], idx_v)

        def body(j, carry):
            o0 = 2 * j * C
            o1 = o0 + C
            g0 = pltpu.async_copy(
                table_hbm.at[idx_v.at[pl.ds(o0, C)]], buf0, gs0)
            g1 = pltpu.async_copy(
                table_hbm.at[idx_v.at[pl.ds(o1, C)]], buf1, gs1)
            g0.wait()
            w0 = pltpu.async_copy(buf0, out_hbm.at[pl.ds(base + o0, C)], ws0)
            g1.wait()
            w1 = pltpu.async_copy(buf1, out_hbm.at[pl.ds(base + o1, C)], ws1)
            w0.wait()
            w1.wait()
            return carry

        lax.fori_loop(0, nch // 2, body, 0)

    return gather_kernel


def _gather_rows(table, idx, C):
    return _make_sc_gather(table.shape[0], idx.shape[0], C,
                           table.shape[1])(table, idx)


def _proj_body(e_ref, w1_ref, b1_ref, w2_ref, b2_ref, w3_ref, b3_ref, o_ref):
    h = jnp.dot(e_ref[...], w1_ref[...], preferred_element_type=jnp.float32)
    h = jnp.maximum(h + b1_ref[...], 0.0)
    h = jnp.dot(h, w2_ref[...], preferred_element_type=jnp.float32)
    h = jnp.maximum(h + b2_ref[...], 0.0)
    h = jnp.dot(h, w3_ref[...], preferred_element_type=jnp.float32)
    o_ref[...] = h + b3_ref[...]


def _proj_call(e, w1t, b1, w2t, b2, w3t, b3):
    TN = 1000
    nt = N // TN
    const = lambda shape: pl.BlockSpec(shape, lambda i: (0, 0))
    return pl.pallas_call(
        _proj_body,
        grid=(nt,),
        in_specs=[
            pl.BlockSpec((TN, NF * D), lambda i: (i, 0)),
            const(w1t.shape), const(b1.shape),
            const(w2t.shape), const(b2.shape),
            const(w3t.shape), const(b3.shape),
        ],
        out_specs=pl.BlockSpec((TN, D), lambda i: (i, 0)),
        out_shape=jax.ShapeDtypeStruct((N, D), jnp.float32),
    )(e, w1t, b1, w2t, b2, w3t, b3)


def _lstm_scan(m_ref, a1, a2, b2, tn):
    def step(t, hc):
        h, c = hc
        gates = (
            jnp.dot(m_ref[t], a1, preferred_element_type=jnp.float32)
            + jnp.dot(h.astype(jnp.bfloat16), a2,
                      preferred_element_type=jnp.float32)
            + b2
        )
        i = jax.nn.sigmoid(gates[:, 0 * D:1 * D])
        f = jax.nn.sigmoid(gates[:, 1 * D:2 * D])
        g = jnp.tanh(gates[:, 2 * D:3 * D])
        o = jax.nn.sigmoid(gates[:, 3 * D:4 * D])
        c = f * c + i * g
        h = o * jnp.tanh(c)
        return (h, c)

    z = jnp.zeros((tn, D), jnp.float32)
    h, _ = lax.fori_loop(0, DEG, step, (z, z))
    return h


def _make_layer_body(tn):
    def body(h_ref, mf_ref, mb_ref, mm_ref,
             a1f_ref, a2f_ref, b2f_ref, wnf_ref,
             a1b_ref, a2b_ref, b2b_ref, wnb_ref,
             wnm_ref, wss_ref, bss_ref, g_ref, bln_ref, o_ref):
        x = h_ref[...]
        hf = _lstm_scan(mf_ref, a1f_ref[...], a2f_ref[...], b2f_ref[...], tn)
        hb = _lstm_scan(mb_ref, a1b_ref[...], a2b_ref[...], b2b_ref[...], tn)
        hm = jnp.mean(mm_ref[...].astype(jnp.float32), axis=0)
        out = (
            jnp.dot(x, wss_ref[...], preferred_element_type=jnp.float32)
            + bss_ref[...]
            + jnp.dot(hf, wnf_ref[...], preferred_element_type=jnp.float32)
            + jnp.dot(hb, wnb_ref[...], preferred_element_type=jnp.float32)
            + jnp.dot(hm, wnm_ref[...], preferred_element_type=jnp.float32)
        )
        out = out * (1.0 / 3.0) + x
        mu = jnp.mean(out, axis=1, keepdims=True)
        var = jnp.mean((out - mu) ** 2, axis=1, keepdims=True)
        out = (out - mu) * jax.lax.rsqrt(var + 1e-5) * g_ref[...] + bln_ref[...]
        o_ref[...] = jnp.maximum(out, 0.0)

    return body


def _layer_call(h, mf, mb, mm, w, TN=400):
    nt = N // TN
    mspec = pl.BlockSpec((DEG, TN, D), lambda i: (0, i, 0))
    const = lambda arr: pl.BlockSpec(arr.shape, lambda i: (0, 0))
    (a1f, a2f, b2f, wnf, a1b, a2b, b2b, wnb, wnm, wss, bss, g, bln) = w
    return pl.pallas_call(
        _make_layer_body(TN),
        grid=(nt,),
        in_specs=[
            pl.BlockSpec((TN, D), lambda i: (i, 0)),
            mspec, mspec, mspec,
            const(a1f), const(a2f), const(b2f), const(wnf),
            const(a1b), const(a2b), const(b2b), const(wnb),
            const(wnm), const(wss), const(bss), const(g), const(bln),
        ],
        out_specs=pl.BlockSpec((TN, D), lambda i: (i, 0)),
        out_shape=jax.ShapeDtypeStruct((N, D), jnp.float32),
    )(h, mf, mb, mm, a1f, a2f, b2f, wnf, a1b, a2b, b2b, wnb, wnm, wss, bss,
      g, bln)


def _make_pool_body(nt):
    def body(h_ref, oh_ref, w1_ref, b1_ref, w2_ref, b2_ref, w3_ref, b3_ref,
             o_ref, acc_ref):
        i = pl.program_id(0)

        @pl.when(i == 0)
        def _init():
            acc_ref[...] = jnp.full((G, D), -jnp.inf, jnp.float32)

        h = h_ref[...]
        oh = oh_ref[...]
        for gidx in range(G):
            m = oh[:, gidx:gidx + 1] > 0.5
            vals = jnp.where(m, h, -jnp.inf)
            acc_ref[pl.ds(gidx, 1), :] = jnp.maximum(
                acc_ref[pl.ds(gidx, 1), :],
                jnp.max(vals, axis=0, keepdims=True))

        @pl.when(i == nt - 1)
        def _head():
            z = jnp.dot(acc_ref[...], w1_ref[...],
                        preferred_element_type=jnp.float32)
            z = jnp.maximum(z + b1_ref[...], 0.0)
            z = jnp.dot(z, w2_ref[...], preferred_element_type=jnp.float32)
            z = jnp.maximum(z + b2_ref[...], 0.0)
            z = jnp.dot(z, w3_ref[...], preferred_element_type=jnp.float32)
            o_ref[...] = z + b3_ref[...]

    return body


def _pool_call(h, onehot, w1t, b1, w2t, b2, w3t, b3):
    TN = 1000
    nt = N // TN
    const = lambda arr: pl.BlockSpec(arr.shape, lambda i: (0, 0))
    return pl.pallas_call(
        _make_pool_body(nt),
        grid=(nt,),
        in_specs=[
            pl.BlockSpec((TN, D), lambda i: (i, 0)),
            pl.BlockSpec((TN, G), lambda i: (i, 0)),
            const(w1t), const(b1), const(w2t), const(b2), const(w3t),
            const(b3),
        ],
        out_specs=pl.BlockSpec((G, NCLS), lambda i: (0, 0)),
        out_shape=jax.ShapeDtypeStruct((G, NCLS), jnp.float32),
        scratch_shapes=[pltpu.VMEM((G, D), jnp.float32)],
    )(h, onehot, w1t, b1, w2t, b2, w3t, b3)


def _tmajor(src):
    return src.reshape(N, DEG).T.reshape(-1).astype(jnp.int32)


def kernel(params, feat_ids, src_forward, src_backward, src_repeat_next,
           graph_ids):
    p = params
    f32 = jnp.float32

    table = jnp.concatenate([p["emb_%d" % i] for i in range(NF)], axis=0)
    offs = (jnp.arange(NF, dtype=jnp.int32) * (VOCAB + 1))[None, :]
    idx_emb = (feat_ids.astype(jnp.int32) + offs).reshape(-1)
    B_emb = 40960
    idx_emb = jnp.concatenate(
        [idx_emb, jnp.zeros((B_emb - N * NF,), jnp.int32)])
    emb_rows = _gather_rows(table, idx_emb, C=128)
    e = emb_rows[: N * NF].reshape(N, NF * D)

    h = _proj_call(
        e,
        p["proj_W1"].T, p["proj_b1"].reshape(1, -1).astype(f32),
        p["proj_W2"].T, p["proj_b2"].reshape(1, -1).astype(f32),
        p["proj_W3"].T, p["proj_b3"].reshape(1, -1).astype(f32),
    )

    idx_all = jnp.concatenate(
        [_tmajor(src_forward), _tmajor(src_backward),
         _tmajor(src_repeat_next)])

    bf16 = jnp.bfloat16
    for l in range(2):
        h_pk = lax.bitcast_convert_type(
            h.astype(bf16).reshape(N, D // 2, 2), jnp.float32)
        rows = _gather_rows(h_pk, idx_all, C=600)
        rows = lax.bitcast_convert_type(rows, bf16).reshape(3 * N * DEG, D)
        mf = rows[0 * N * DEG: 1 * N * DEG].reshape(DEG, N, D)
        mb = rows[1 * N * DEG: 2 * N * DEG].reshape(DEG, N, D)
        mm = rows[2 * N * DEG: 3 * N * DEG].reshape(DEG, N, D)
        pre = "l%d_" % l
        wss = (p[pre + "forward_Wself"] + p[pre + "backward_Wself"]
               + p[pre + "repeat_next_Wself"]).T
        bss = (p[pre + "forward_bself"] + p[pre + "backward_bself"]
               + p[pre + "repeat_next_bself"]).reshape(1, -1).astype(f32)
        w = (
            p[pre + "forward_Wih"].T.astype(bf16),
            p[pre + "forward_Whh"].T.astype(bf16),
            (p[pre + "forward_bih"] + p[pre + "forward_bhh"]
             ).reshape(1, -1).astype(f32),
            p[pre + "forward_Wneigh"].T,
            p[pre + "backward_Wih"].T.astype(bf16),
            p[pre + "backward_Whh"].T.astype(bf16),
            (p[pre + "backward_bih"] + p[pre + "backward_bhh"]
             ).reshape(1, -1).astype(f32),
            p[pre + "backward_Wneigh"].T,
            p[pre + "repeat_next_Wneigh"].T,
            wss,
            bss,
            p["ln%d_g" % l].reshape(1, -1).astype(f32),
            p["ln%d_b" % l].reshape(1, -1).astype(f32),
        )
        h = _layer_call(h, mf, mb, mm, w)

    onehot = (graph_ids[:, None] == jnp.arange(G, dtype=graph_ids.dtype)
              [None, :]).astype(f32)
    return _pool_call(
        h, onehot,
        p["cls_W1"].T, p["cls_b1"].reshape(1, -1).astype(f32),
        p["cls_W2"].T, p["cls_b2"].reshape(1, -1).astype(f32),
        p["cls_W3"].T, p["cls_b3"].reshape(1, -1).astype(f32),
    )


# speedup vs baseline: 2.6128x; 2.6128x over previous
import functools

import jax
import jax.numpy as jnp
from jax import lax
from jax.experimental import pallas as pl
from jax.experimental.pallas import tpu as pltpu
from jax.experimental.pallas import tpu_sc as plsc

N = 10000
DEG = 32
D = 128
NF = 4
VOCAB = 1000
G = 16
NCLS = 33

_SC_NC = 2
_SC_NS = 16
_SC_NW = _SC_NC * _SC_NS


@functools.lru_cache(maxsize=None)
def _make_sc_gather(V, B, C, W=D):
    del V
    bpw = B // _SC_NW
    assert B % _SC_NW == 0 and bpw % C == 0 and C % 8 == 0
    nch = bpw // C
    assert nch % 2 == 0
    mesh = plsc.VectorSubcoreMesh(core_axis_name="c", subcore_axis_name="s")

    @functools.partial(
        pl.kernel,
        mesh=mesh,
        out_type=jax.ShapeDtypeStruct((B, W), jnp.float32),
        scratch_types=[
            pltpu.VMEM((bpw,), jnp.int32),
            pltpu.VMEM((C, W), jnp.float32),
            pltpu.VMEM((C, W), jnp.float32),
            pltpu.SemaphoreType.DMA,
            pltpu.SemaphoreType.DMA,
            pltpu.SemaphoreType.DMA,
            pltpu.SemaphoreType.DMA,
        ],
    )
    def gather_kernel(table_hbm, idx_hbm, out_hbm, idx_v, buf0, buf1,
                      gs0, gs1, ws0, ws1):
        wid = lax.axis_index("s") * _SC_NC + lax.axis_index("c")
        base = wid * bpw
        pltpu.sync_copy(idx_hbm.at[pl.ds(base, bpw)], idx_v)

        def body(j, carry):
            o0 = 2 * j * C
            o1 = o0 + C
            g0 = pltpu.async_copy(
                table_hbm.at[idx_v.at[pl.ds(o0, C)]], buf0, gs0)
            g1 = pltpu.async_copy(
                table_hbm.at[idx_v.at[pl.ds(o1, C)]], buf1, gs1)
            g0.wait()
            w0 = pltpu.async_copy(buf0, out_hbm.at[pl.ds(base + o0, C)], ws0)
            g1.wait()
            w1 = pltpu.async_copy(buf1, out_hbm.at[pl.ds(base + o1, C)], ws1)
            w0.wait()
            w1.wait()
            return carry

        lax.fori_loop(0, nch // 2, body, 0)

    return gather_kernel


def _gather_rows(table, idx, C):
    return _make_sc_gather(table.shape[0], idx.shape[0], C,
                           table.shape[1])(table, idx)


def _proj_body(e_ref, w1_ref, b1_ref, w2_ref, b2_ref, w3_ref, b3_ref, o_ref):
    h = jnp.dot(e_ref[...], w1_ref[...], preferred_element_type=jnp.float32)
    h = jnp.maximum(h + b1_ref[...], 0.0)
    h = jnp.dot(h, w2_ref[...], preferred_element_type=jnp.float32)
    h = jnp.maximum(h + b2_ref[...], 0.0)
    h = jnp.dot(h, w3_ref[...], preferred_element_type=jnp.float32)
    o_ref[...] = h + b3_ref[...]


def _proj_call(e, w1t, b1, w2t, b2, w3t, b3):
    TN = 1000
    nt = N // TN
    const = lambda shape: pl.BlockSpec(shape, lambda i: (0, 0))
    return pl.pallas_call(
        _proj_body,
        grid=(nt,),
        in_specs=[
            pl.BlockSpec((TN, NF * D), lambda i: (i, 0)),
            const(w1t.shape), const(b1.shape),
            const(w2t.shape), const(b2.shape),
            const(w3t.shape), const(b3.shape),
        ],
        out_specs=pl.BlockSpec((TN, D), lambda i: (i, 0)),
        out_shape=jax.ShapeDtypeStruct((N, D), jnp.float32),
    )(e, w1t, b1, w2t, b2, w3t, b3)


def _lstm_scan(m_ref, a1, a2, b2, tn):
    def step(t, hc):
        h, c = hc
        gates = (
            jnp.dot(m_ref[t].astype(jnp.bfloat16), a1,
                    preferred_element_type=jnp.float32)
            + jnp.dot(h.astype(jnp.bfloat16), a2,
                      preferred_element_type=jnp.float32)
            + b2
        )
        i = jax.nn.sigmoid(gates[:, 0 * D:1 * D])
        f = jax.nn.sigmoid(gates[:, 1 * D:2 * D])
        g = jnp.tanh(gates[:, 2 * D:3 * D])
        o = jax.nn.sigmoid(gates[:, 3 * D:4 * D])
        c = f * c + i * g
        h = o * jnp.tanh(c)
        return (h, c)

    z = jnp.zeros((tn, D), jnp.float32)
    h, _ = lax.fori_loop(0, DEG, step, (z, z))
    return h


def _make_layer_body(tn):
    def body(h_ref, mf_ref, mb_ref, mm_ref,
             a1f_ref, a2f_ref, b2f_ref, wnf_ref,
             a1b_ref, a2b_ref, b2b_ref, wnb_ref,
             wnm_ref, wss_ref, bss_ref, g_ref, bln_ref, o_ref):
        x = h_ref[...]
        hf = _lstm_scan(mf_ref, a1f_ref[...], a2f_ref[...], b2f_ref[...], tn)
        hb = _lstm_scan(mb_ref, a1b_ref[...], a2b_ref[...], b2b_ref[...], tn)
        hm = jnp.mean(mm_ref[...].astype(jnp.float32), axis=0)
        out = (
            jnp.dot(x, wss_ref[...], preferred_element_type=jnp.float32)
            + bss_ref[...]
            + jnp.dot(hf, wnf_ref[...], preferred_element_type=jnp.float32)
            + jnp.dot(hb, wnb_ref[...], preferred_element_type=jnp.float32)
            + jnp.dot(hm, wnm_ref[...], preferred_element_type=jnp.float32)
        )
        out = out * (1.0 / 3.0) + x
        mu = jnp.mean(out, axis=1, keepdims=True)
        var = jnp.mean((out - mu) ** 2, axis=1, keepdims=True)
        out = (out - mu) * jax.lax.rsqrt(var + 1e-5) * g_ref[...] + bln_ref[...]
        o_ref[...] = jnp.maximum(out, 0.0)

    return body


def _layer_call(h, mf, mb, mm, w, TN=400):
    nt = N // TN
    mspec = pl.BlockSpec((DEG, TN, D), lambda i: (0, i, 0))
    const = lambda arr: pl.BlockSpec(arr.shape, lambda i: (0, 0))
    (a1f, a2f, b2f, wnf, a1b, a2b, b2b, wnb, wnm, wss, bss, g, bln) = w
    return pl.pallas_call(
        _make_layer_body(TN),
        grid=(nt,),
        in_specs=[
            pl.BlockSpec((TN, D), lambda i: (i, 0)),
            mspec, mspec, mspec,
            const(a1f), const(a2f), const(b2f), const(wnf),
            const(a1b), const(a2b), const(b2b), const(wnb),
            const(wnm), const(wss), const(bss), const(g), const(bln),
        ],
        out_specs=pl.BlockSpec((TN, D), lambda i: (i, 0)),
        out_shape=jax.ShapeDtypeStruct((N, D), jnp.float32),
    )(h, mf, mb, mm, a1f, a2f, b2f, wnf, a1b, a2b, b2b, wnb, wnm, wss, bss,
      g, bln)


def _make_pool_body(nt):
    def body(h_ref, oh_ref, w1_ref, b1_ref, w2_ref, b2_ref, w3_ref, b3_ref,
             o_ref, acc_ref):
        i = pl.program_id(0)

        @pl.when(i == 0)
        def _init():
            acc_ref[...] = jnp.full((G, D), -jnp.inf, jnp.float32)

        h = h_ref[...]
        oh = oh_ref[...]
        for gidx in range(G):
            m = oh[:, gidx:gidx + 1] > 0.5
            vals = jnp.where(m, h, -jnp.inf)
            acc_ref[pl.ds(gidx, 1), :] = jnp.maximum(
                acc_ref[pl.ds(gidx, 1), :],
                jnp.max(vals, axis=0, keepdims=True))

        @pl.when(i == nt - 1)
        def _head():
            z = jnp.dot(acc_ref[...], w1_ref[...],
                        preferred_element_type=jnp.float32)
            z = jnp.maximum(z + b1_ref[...], 0.0)
            z = jnp.dot(z, w2_ref[...], preferred_element_type=jnp.float32)
            z = jnp.maximum(z + b2_ref[...], 0.0)
            z = jnp.dot(z, w3_ref[...], preferred_element_type=jnp.float32)
            o_ref[...] = z + b3_ref[...]

    return body


def _pool_call(h, onehot, w1t, b1, w2t, b2, w3t, b3):
    TN = 1000
    nt = N // TN
    const = lambda arr: pl.BlockSpec(arr.shape, lambda i: (0, 0))
    return pl.pallas_call(
        _make_pool_body(nt),
        grid=(nt,),
        in_specs=[
            pl.BlockSpec((TN, D), lambda i: (i, 0)),
            pl.BlockSpec((TN, G), lambda i: (i, 0)),
            const(w1t), const(b1), const(w2t), const(b2), const(w3t),
            const(b3),
        ],
        out_specs=pl.BlockSpec((G, NCLS), lambda i: (0, 0)),
        out_shape=jax.ShapeDtypeStruct((G, NCLS), jnp.float32),
        scratch_shapes=[pltpu.VMEM((G, D), jnp.float32)],
    )(h, onehot, w1t, b1, w2t, b2, w3t, b3)


def _tmajor(src):
    return src.reshape(N, DEG).T.reshape(-1).astype(jnp.int32)


def kernel(params, feat_ids, src_forward, src_backward, src_repeat_next,
           graph_ids):
    p = params
    f32 = jnp.float32

    table = jnp.concatenate([p["emb_%d" % i] for i in range(NF)], axis=0)
    offs = (jnp.arange(NF, dtype=jnp.int32) * (VOCAB + 1))[None, :]
    idx_emb = (feat_ids.astype(jnp.int32) + offs).reshape(-1)
    B_emb = 40960
    idx_emb = jnp.concatenate(
        [idx_emb, jnp.zeros((B_emb - N * NF,), jnp.int32)])
    emb_rows = _gather_rows(table, idx_emb, C=128)
    e = emb_rows[: N * NF].reshape(N, NF * D)

    h = _proj_call(
        e,
        p["proj_W1"].T, p["proj_b1"].reshape(1, -1).astype(f32),
        p["proj_W2"].T, p["proj_b2"].reshape(1, -1).astype(f32),
        p["proj_W3"].T, p["proj_b3"].reshape(1, -1).astype(f32),
    )

    idx_all = jnp.concatenate(
        [_tmajor(src_forward), _tmajor(src_backward),
         _tmajor(src_repeat_next)])

    bf16 = jnp.bfloat16
    for l in range(2):
        rows = _gather_rows(h, idx_all, C=200)
        mf = rows[0 * N * DEG: 1 * N * DEG].reshape(DEG, N, D)
        mb = rows[1 * N * DEG: 2 * N * DEG].reshape(DEG, N, D)
        mm = rows[2 * N * DEG: 3 * N * DEG].reshape(DEG, N, D)
        pre = "l%d_" % l
        wss = (p[pre + "forward_Wself"] + p[pre + "backward_Wself"]
               + p[pre + "repeat_next_Wself"]).T
        bss = (p[pre + "forward_bself"] + p[pre + "backward_bself"]
               + p[pre + "repeat_next_bself"]).reshape(1, -1).astype(f32)
        w = (
            p[pre + "forward_Wih"].T.astype(bf16),
            p[pre + "forward_Whh"].T.astype(bf16),
            (p[pre + "forward_bih"] + p[pre + "forward_bhh"]
             ).reshape(1, -1).astype(f32),
            p[pre + "forward_Wneigh"].T,
            p[pre + "backward_Wih"].T.astype(bf16),
            p[pre + "backward_Whh"].T.astype(bf16),
            (p[pre + "backward_bih"] + p[pre + "backward_bhh"]
             ).reshape(1, -1).astype(f32),
            p[pre + "backward_Wneigh"].T,
            p[pre + "repeat_next_Wneigh"].T,
            wss,
            bss,
            p["ln%d_g" % l].reshape(1, -1).astype(f32),
            p["ln%d_b" % l].reshape(1, -1).astype(f32),
        )
        h = _layer_call(h, mf, mb, mm, w)

    onehot = (graph_ids[:, None] == jnp.arange(G, dtype=graph_ids.dtype)
              [None, :]).astype(f32)
    return _pool_call(
        h, onehot,
        p["cls_W1"].T, p["cls_b1"].reshape(1, -1).astype(f32),
        p["cls_W2"].T, p["cls_b2"].reshape(1, -1).astype(f32),
        p["cls_W3"].T, p["cls_b3"].reshape(1, -1).astype(f32),
    )


# speedup vs baseline: 3.7826x; 1.4477x over previous
import functools

import jax
import jax.numpy as jnp
from jax import lax
from jax.experimental import pallas as pl
from jax.experimental.pallas import tpu as pltpu
from jax.experimental.pallas import tpu_sc as plsc

N = 10000
DEG = 32
D = 128
NF = 4
VOCAB = 1000
G = 16
NCLS = 33

_SC_NC = 2
_SC_NS = 16
_SC_NW = _SC_NC * _SC_NS


@functools.lru_cache(maxsize=None)
def _make_sc_gather(V, B, C, W=D):
    del V
    bpw = B // _SC_NW
    assert B % _SC_NW == 0 and bpw % C == 0 and C % 8 == 0
    nch = bpw // C
    assert nch % 2 == 0
    mesh = plsc.VectorSubcoreMesh(core_axis_name="c", subcore_axis_name="s")

    @functools.partial(
        pl.kernel,
        mesh=mesh,
        out_type=jax.ShapeDtypeStruct((B, W), jnp.float32),
        scratch_types=[
            pltpu.VMEM((bpw,), jnp.int32),
            pltpu.VMEM((C, W), jnp.float32),
            pltpu.VMEM((C, W), jnp.float32),
            pltpu.SemaphoreType.DMA,
            pltpu.SemaphoreType.DMA,
            pltpu.SemaphoreType.DMA,
            pltpu.SemaphoreType.DMA,
        ],
    )
    def gather_kernel(table_hbm, idx_hbm, out_hbm, idx_v, buf0, buf1,
                      gs0, gs1, ws0, ws1):
        wid = lax.axis_index("s") * _SC_NC + lax.axis_index("c")
        base = wid * bpw
        pltpu.sync_copy(idx_hbm.at[pl.ds(base, bpw)], idx_v)

        def body(j, carry):
            o0 = 2 * j * C
            o1 = o0 + C
            g0 = pltpu.async_copy(
                table_hbm.at[idx_v.at[pl.ds(o0, C)]], buf0, gs0)
            g1 = pltpu.async_copy(
                table_hbm.at[idx_v.at[pl.ds(o1, C)]], buf1, gs1)
            g0.wait()
            w0 = pltpu.async_copy(buf0, out_hbm.at[pl.ds(base + o0, C)], ws0)
            g1.wait()
            w1 = pltpu.async_copy(buf1, out_hbm.at[pl.ds(base + o1, C)], ws1)
            w0.wait()
            w1.wait()
            return carry

        lax.fori_loop(0, nch // 2, body, 0)

    return gather_kernel


def _gather_rows(table, idx, C):
    return _make_sc_gather(table.shape[0], idx.shape[0], C,
                           table.shape[1])(table, idx)


def _proj_body(e_ref, w1_ref, b1_ref, w2_ref, b2_ref, w3_ref, b3_ref, o_ref):
    h = jnp.dot(e_ref[...], w1_ref[...], preferred_element_type=jnp.float32)
    h = jnp.maximum(h + b1_ref[...], 0.0)
    h = jnp.dot(h, w2_ref[...], preferred_element_type=jnp.float32)
    h = jnp.maximum(h + b2_ref[...], 0.0)
    h = jnp.dot(h, w3_ref[...], preferred_element_type=jnp.float32)
    o_ref[...] = h + b3_ref[...]


def _proj_call(e, w1t, b1, w2t, b2, w3t, b3):
    TN = 1000
    nt = N // TN
    const = lambda shape: pl.BlockSpec(shape, lambda i: (0, 0))
    return pl.pallas_call(
        _proj_body,
        grid=(nt,),
        in_specs=[
            pl.BlockSpec((TN, NF * D), lambda i: (i, 0)),
            const(w1t.shape), const(b1.shape),
            const(w2t.shape), const(b2.shape),
            const(w3t.shape), const(b3.shape),
        ],
        out_specs=pl.BlockSpec((TN, D), lambda i: (i, 0)),
        out_shape=jax.ShapeDtypeStruct((N, D), jnp.float32),
    )(e, w1t, b1, w2t, b2, w3t, b3)


def _lstm_scan(m_ref, a1, a2, b2, tn):
    def step(t, hc):
        h, c = hc
        gates = (
            jnp.dot(m_ref[t].astype(jnp.bfloat16), a1,
                    preferred_element_type=jnp.float32)
            + jnp.dot(h.astype(jnp.bfloat16), a2,
                      preferred_element_type=jnp.float32)
            + b2
        )
        i = jax.nn.sigmoid(gates[:, 0 * D:1 * D])
        f = jax.nn.sigmoid(gates[:, 1 * D:2 * D])
        g = jnp.tanh(gates[:, 2 * D:3 * D])
        o = jax.nn.sigmoid(gates[:, 3 * D:4 * D])
        c = f * c + i * g
        h = o * jnp.tanh(c)
        return (h, c)

    z = jnp.zeros((tn, D), jnp.float32)
    h, _ = lax.fori_loop(0, DEG, step, (z, z))
    return h


def _make_lstm_body(tn):
    def body(m_ref, a1_ref, a2_ref, b2_ref, o_ref):
        o_ref[...] = _lstm_scan(m_ref, a1_ref[...], a2_ref[...], b2_ref[...],
                                tn)

    return body


def _lstm_call(m, a1, a2, b2, TN=400):
    nt = N // TN
    const = lambda arr: pl.BlockSpec(arr.shape, lambda i: (0, 0))
    return pl.pallas_call(
        _make_lstm_body(TN),
        grid=(nt,),
        in_specs=[
            pl.BlockSpec((DEG, TN, D), lambda i: (0, i, 0)),
            const(a1), const(a2), const(b2),
        ],
        out_specs=pl.BlockSpec((TN, D), lambda i: (i, 0)),
        out_shape=jax.ShapeDtypeStruct((N, D), jnp.float32),
    )(m, a1, a2, b2)


def _combine_body(h_ref, hf_ref, hb_ref, mm_ref,
                  wnf_ref, wnb_ref, wnm_ref, wss_ref, bss_ref,
                  g_ref, bln_ref, o_ref):
    x = h_ref[...]
    hm = jnp.mean(mm_ref[...].astype(jnp.float32), axis=0)
    out = (
        jnp.dot(x, wss_ref[...], preferred_element_type=jnp.float32)
        + bss_ref[...]
        + jnp.dot(hf_ref[...], wnf_ref[...],
                  preferred_element_type=jnp.float32)
        + jnp.dot(hb_ref[...], wnb_ref[...],
                  preferred_element_type=jnp.float32)
        + jnp.dot(hm, wnm_ref[...], preferred_element_type=jnp.float32)
    )
    out = out * (1.0 / 3.0) + x
    mu = jnp.mean(out, axis=1, keepdims=True)
    var = jnp.mean((out - mu) ** 2, axis=1, keepdims=True)
    out = (out - mu) * jax.lax.rsqrt(var + 1e-5) * g_ref[...] + bln_ref[...]
    o_ref[...] = jnp.maximum(out, 0.0)


def _combine_call(h, hf, hb, mm, wnf, wnb, wnm, wss, bss, g, bln, TN=1000):
    nt = N // TN
    const = lambda arr: pl.BlockSpec(arr.shape, lambda i: (0, 0))
    nspec = pl.BlockSpec((TN, D), lambda i: (i, 0))
    return pl.pallas_call(
        _combine_body,
        grid=(nt,),
        in_specs=[
            nspec, nspec, nspec,
            pl.BlockSpec((DEG, TN, D), lambda i: (0, i, 0)),
            const(wnf), const(wnb), const(wnm), const(wss), const(bss),
            const(g), const(bln),
        ],
        out_specs=nspec,
        out_shape=jax.ShapeDtypeStruct((N, D), jnp.float32),
    )(h, hf, hb, mm, wnf, wnb, wnm, wss, bss, g, bln)


def _make_pool_body(nt):
    def body(h_ref, oh_ref, w1_ref, b1_ref, w2_ref, b2_ref, w3_ref, b3_ref,
             o_ref, acc_ref):
        i = pl.program_id(0)

        @pl.when(i == 0)
        def _init():
            acc_ref[...] = jnp.full((G, D), -jnp.inf, jnp.float32)

        h = h_ref[...]
        oh = oh_ref[...]
        for gidx in range(G):
            m = oh[:, gidx:gidx + 1] > 0.5
            vals = jnp.where(m, h, -jnp.inf)
            acc_ref[pl.ds(gidx, 1), :] = jnp.maximum(
                acc_ref[pl.ds(gidx, 1), :],
                jnp.max(vals, axis=0, keepdims=True))

        @pl.when(i == nt - 1)
        def _head():
            z = jnp.dot(acc_ref[...], w1_ref[...],
                        preferred_element_type=jnp.float32)
            z = jnp.maximum(z + b1_ref[...], 0.0)
            z = jnp.dot(z, w2_ref[...], preferred_element_type=jnp.float32)
            z = jnp.maximum(z + b2_ref[...], 0.0)
            z = jnp.dot(z, w3_ref[...], preferred_element_type=jnp.float32)
            o_ref[...] = z + b3_ref[...]

    return body


def _pool_call(h, onehot, w1t, b1, w2t, b2, w3t, b3):
    TN = 1000
    nt = N // TN
    const = lambda arr: pl.BlockSpec(arr.shape, lambda i: (0, 0))
    return pl.pallas_call(
        _make_pool_body(nt),
        grid=(nt,),
        in_specs=[
            pl.BlockSpec((TN, D), lambda i: (i, 0)),
            pl.BlockSpec((TN, G), lambda i: (i, 0)),
            const(w1t), const(b1), const(w2t), const(b2), const(w3t),
            const(b3),
        ],
        out_specs=pl.BlockSpec((G, NCLS), lambda i: (0, 0)),
        out_shape=jax.ShapeDtypeStruct((G, NCLS), jnp.float32),
        scratch_shapes=[pltpu.VMEM((G, D), jnp.float32)],
    )(h, onehot, w1t, b1, w2t, b2, w3t, b3)


def _tmajor(src):
    return src.reshape(N, DEG).T.reshape(-1).astype(jnp.int32)


def kernel(params, feat_ids, src_forward, src_backward, src_repeat_next,
           graph_ids):
    p = params
    f32 = jnp.float32

    table = jnp.concatenate([p["emb_%d" % i] for i in range(NF)], axis=0)
    offs = (jnp.arange(NF, dtype=jnp.int32) * (VOCAB + 1))[None, :]
    idx_emb = (feat_ids.astype(jnp.int32) + offs).reshape(-1)
    B_emb = 40960
    idx_emb = jnp.concatenate(
        [idx_emb, jnp.zeros((B_emb - N * NF,), jnp.int32)])
    emb_rows = _gather_rows(table, idx_emb, C=128)
    e = emb_rows[: N * NF].reshape(N, NF * D)

    h = _proj_call(
        e,
        p["proj_W1"].T, p["proj_b1"].reshape(1, -1).astype(f32),
        p["proj_W2"].T, p["proj_b2"].reshape(1, -1).astype(f32),
        p["proj_W3"].T, p["proj_b3"].reshape(1, -1).astype(f32),
    )

    idx_f = _tmajor(src_forward)
    idx_b = _tmajor(src_backward)
    idx_m = _tmajor(src_repeat_next)

    bf16 = jnp.bfloat16
    for l in range(2):
        rf = _gather_rows(h, idx_f, C=200).reshape(DEG, N, D)
        rb = _gather_rows(h, idx_b, C=200).reshape(DEG, N, D)
        rm = _gather_rows(h, idx_m, C=200).reshape(DEG, N, D)
        pre = "l%d_" % l
        hf = _lstm_call(
            rf,
            p[pre + "forward_Wih"].T.astype(bf16),
            p[pre + "forward_Whh"].T.astype(bf16),
            (p[pre + "forward_bih"] + p[pre + "forward_bhh"]
             ).reshape(1, -1).astype(f32))
        hb = _lstm_call(
            rb,
            p[pre + "backward_Wih"].T.astype(bf16),
            p[pre + "backward_Whh"].T.astype(bf16),
            (p[pre + "backward_bih"] + p[pre + "backward_bhh"]
             ).reshape(1, -1).astype(f32))
        wss = (p[pre + "forward_Wself"] + p[pre + "backward_Wself"]
               + p[pre + "repeat_next_Wself"]).T
        bss = (p[pre + "forward_bself"] + p[pre + "backward_bself"]
               + p[pre + "repeat_next_bself"]).reshape(1, -1).astype(f32)
        h = _combine_call(
            h, hf, hb, rm,
            p[pre + "forward_Wneigh"].T,
            p[pre + "backward_Wneigh"].T,
            p[pre + "repeat_next_Wneigh"].T,
            wss, bss,
            p["ln%d_g" % l].reshape(1, -1).astype(f32),
            p["ln%d_b" % l].reshape(1, -1).astype(f32))

    onehot = (graph_ids[:, None] == jnp.arange(G, dtype=graph_ids.dtype)
              [None, :]).astype(f32)
    return _pool_call(
        h, onehot,
        p["cls_W1"].T, p["cls_b1"].reshape(1, -1).astype(f32),
        p["cls_W2"].T, p["cls_b2"].reshape(1, -1).astype(f32),
        p["cls_W3"].T, p["cls_b3"].reshape(1, -1).astype(f32),
    )


# speedup vs baseline: 4.3711x; 1.1556x over previous
import functools

import jax
import jax.numpy as jnp
from jax import lax
from jax.experimental import pallas as pl
from jax.experimental.pallas import tpu as pltpu
from jax.experimental.pallas import tpu_sc as plsc

N = 10000
DEG = 32
D = 128
NF = 4
VOCAB = 1000
G = 16
NCLS = 33

_SC_NC = 2
_SC_NS = 16
_SC_NW = _SC_NC * _SC_NS


@functools.lru_cache(maxsize=None)
def _make_sc_gather(V, B, C, W=D):
    del V
    bpw = B // _SC_NW
    assert B % _SC_NW == 0 and bpw % C == 0 and C % 8 == 0
    nch = bpw // C
    assert nch % 2 == 0
    mesh = plsc.VectorSubcoreMesh(core_axis_name="c", subcore_axis_name="s")

    @functools.partial(
        pl.kernel,
        mesh=mesh,
        out_type=jax.ShapeDtypeStruct((B, W), jnp.float32),
        scratch_types=[
            pltpu.VMEM((bpw,), jnp.int32),
            pltpu.VMEM((C, W), jnp.float32),
            pltpu.VMEM((C, W), jnp.float32),
            pltpu.SemaphoreType.DMA,
            pltpu.SemaphoreType.DMA,
            pltpu.SemaphoreType.DMA,
            pltpu.SemaphoreType.DMA,
        ],
    )
    def gather_kernel(table_hbm, idx_hbm, out_hbm, idx_v, buf0, buf1,
                      gs0, gs1, ws0, ws1):
        wid = lax.axis_index("s") * _SC_NC + lax.axis_index("c")
        base = wid * bpw
        pltpu.sync_copy(idx_hbm.at[pl.ds(base, bpw)], idx_v)

        def body(j, carry):
            o0 = 2 * j * C
            o1 = o0 + C
            g0 = pltpu.async_copy(
                table_hbm.at[idx_v.at[pl.ds(o0, C)]], buf0, gs0)
            g1 = pltpu.async_copy(
                table_hbm.at[idx_v.at[pl.ds(o1, C)]], buf1, gs1)
            g0.wait()
            w0 = pltpu.async_copy(buf0, out_hbm.at[pl.ds(base + o0, C)], ws0)
            g1.wait()
            w1 = pltpu.async_copy(buf1, out_hbm.at[pl.ds(base + o1, C)], ws1)
            w0.wait()
            w1.wait()
            return carry

        lax.fori_loop(0, nch // 2, body, 0)

    return gather_kernel


def _gather_rows(table, idx, C):
    return _make_sc_gather(table.shape[0], idx.shape[0], C,
                           table.shape[1])(table, idx)


def _proj_body(e_ref, w1_ref, b1_ref, w2_ref, b2_ref, w3_ref, b3_ref, o_ref):
    h = jnp.dot(e_ref[...], w1_ref[...], preferred_element_type=jnp.float32)
    h = jnp.maximum(h + b1_ref[...], 0.0)
    h = jnp.dot(h, w2_ref[...], preferred_element_type=jnp.float32)
    h = jnp.maximum(h + b2_ref[...], 0.0)
    h = jnp.dot(h, w3_ref[...], preferred_element_type=jnp.float32)
    o_ref[...] = h + b3_ref[...]


def _proj_call(e, w1t, b1, w2t, b2, w3t, b3):
    TN = 1000
    nt = N // TN
    const = lambda shape: pl.BlockSpec(shape, lambda i: (0, 0))
    return pl.pallas_call(
        _proj_body,
        grid=(nt,),
        in_specs=[
            pl.BlockSpec((TN, NF * D), lambda i: (i, 0)),
            const(w1t.shape), const(b1.shape),
            const(w2t.shape), const(b2.shape),
            const(w3t.shape), const(b3.shape),
        ],
        out_specs=pl.BlockSpec((TN, D), lambda i: (i, 0)),
        out_shape=jax.ShapeDtypeStruct((N, D), jnp.float32),
    )(e, w1t, b1, w2t, b2, w3t, b3)


def _lstm_scan(m_ref, a1, a2, b2, tn):
    def step(t, hc):
        h, c = hc
        gates = (
            jnp.dot(m_ref[t].astype(jnp.bfloat16), a1,
                    preferred_element_type=jnp.float32)
            + jnp.dot(h.astype(jnp.bfloat16), a2,
                      preferred_element_type=jnp.float32)
            + b2
        )
        i = jax.nn.sigmoid(gates[:, 0 * D:1 * D])
        f = jax.nn.sigmoid(gates[:, 1 * D:2 * D])
        g = jnp.tanh(gates[:, 2 * D:3 * D])
        o = jax.nn.sigmoid(gates[:, 3 * D:4 * D])
        c = f * c + i * g
        h = o * jnp.tanh(c)
        return (h, c)

    z = jnp.zeros((tn, D), jnp.float32)
    h, _ = lax.fori_loop(0, DEG, step, (z, z))
    return h


def _make_lstm_body(tn):
    def body(m_ref, a1_ref, a2_ref, b2_ref, o_ref):
        o_ref[...] = _lstm_scan(m_ref, a1_ref[...], a2_ref[...], b2_ref[...],
                                tn)

    return body


def _lstm_call(m, a1, a2, b2, TN=1000):
    nt = N // TN
    const = lambda arr: pl.BlockSpec(arr.shape, lambda i: (0, 0))
    return pl.pallas_call(
        _make_lstm_body(TN),
        grid=(nt,),
        in_specs=[
            pl.BlockSpec((DEG, TN, D), lambda i: (0, i, 0)),
            const(a1), const(a2), const(b2),
        ],
        out_specs=pl.BlockSpec((TN, D), lambda i: (i, 0)),
        out_shape=jax.ShapeDtypeStruct((N, D), jnp.float32),
    )(m, a1, a2, b2)


def _combine_body(h_ref, hf_ref, hb_ref, mm_ref,
                  wnf_ref, wnb_ref, wnm_ref, wss_ref, bss_ref,
                  g_ref, bln_ref, o_ref):
    x = h_ref[...]
    hm = jnp.mean(mm_ref[...].astype(jnp.float32), axis=0)
    out = (
        jnp.dot(x, wss_ref[...], preferred_element_type=jnp.float32)
        + bss_ref[...]
        + jnp.dot(hf_ref[...], wnf_ref[...],
                  preferred_element_type=jnp.float32)
        + jnp.dot(hb_ref[...], wnb_ref[...],
                  preferred_element_type=jnp.float32)
        + jnp.dot(hm, wnm_ref[...], preferred_element_type=jnp.float32)
    )
    out = out * (1.0 / 3.0) + x
    mu = jnp.mean(out, axis=1, keepdims=True)
    var = jnp.mean((out - mu) ** 2, axis=1, keepdims=True)
    out = (out - mu) * jax.lax.rsqrt(var + 1e-5) * g_ref[...] + bln_ref[...]
    o_ref[...] = jnp.maximum(out, 0.0)


def _combine_call(h, hf, hb, mm, wnf, wnb, wnm, wss, bss, g, bln, TN=1000):
    nt = N // TN
    const = lambda arr: pl.BlockSpec(arr.shape, lambda i: (0, 0))
    nspec = pl.BlockSpec((TN, D), lambda i: (i, 0))
    return pl.pallas_call(
        _combine_body,
        grid=(nt,),
        in_specs=[
            nspec, nspec, nspec,
            pl.BlockSpec((DEG, TN, D), lambda i: (0, i, 0)),
            const(wnf), const(wnb), const(wnm), const(wss), const(bss),
            const(g), const(bln),
        ],
        out_specs=nspec,
        out_shape=jax.ShapeDtypeStruct((N, D), jnp.float32),
    )(h, hf, hb, mm, wnf, wnb, wnm, wss, bss, g, bln)


def _make_pool_body(nt):
    def body(h_ref, oh_ref, w1_ref, b1_ref, w2_ref, b2_ref, w3_ref, b3_ref,
             o_ref, acc_ref):
        i = pl.program_id(0)

        @pl.when(i == 0)
        def _init():
            acc_ref[...] = jnp.full((G, D), -jnp.inf, jnp.float32)

        h = h_ref[...]
        oh = oh_ref[...]
        for gidx in range(G):
            m = oh[:, gidx:gidx + 1] > 0.5
            vals = jnp.where(m, h, -jnp.inf)
            acc_ref[pl.ds(gidx, 1), :] = jnp.maximum(
                acc_ref[pl.ds(gidx, 1), :],
                jnp.max(vals, axis=0, keepdims=True))

        @pl.when(i == nt - 1)
        def _head():
            z = jnp.dot(acc_ref[...], w1_ref[...],
                        preferred_element_type=jnp.float32)
            z = jnp.maximum(z + b1_ref[...], 0.0)
            z = jnp.dot(z, w2_ref[...], preferred_element_type=jnp.float32)
            z = jnp.maximum(z + b2_ref[...], 0.0)
            z = jnp.dot(z, w3_ref[...], preferred_element_type=jnp.float32)
            o_ref[...] = z + b3_ref[...]

    return body


def _pool_call(h, onehot, w1t, b1, w2t, b2, w3t, b3):
    TN = 1000
    nt = N // TN
    const = lambda arr: pl.BlockSpec(arr.shape, lambda i: (0, 0))
    return pl.pallas_call(
        _make_pool_body(nt),
        grid=(nt,),
        in_specs=[
            pl.BlockSpec((TN, D), lambda i: (i, 0)),
            pl.BlockSpec((TN, G), lambda i: (i, 0)),
            const(w1t), const(b1), const(w2t), const(b2), const(w3t),
            const(b3),
        ],
        out_specs=pl.BlockSpec((G, NCLS), lambda i: (0, 0)),
        out_shape=jax.ShapeDtypeStruct((G, NCLS), jnp.float32),
        scratch_shapes=[pltpu.VMEM((G, D), jnp.float32)],
    )(h, onehot, w1t, b1, w2t, b2, w3t, b3)


def _tmajor(src):
    return src.reshape(N, DEG).T.reshape(-1).astype(jnp.int32)


def kernel(params, feat_ids, src_forward, src_backward, src_repeat_next,
           graph_ids):
    p = params
    f32 = jnp.float32

    table = jnp.concatenate([p["emb_%d" % i] for i in range(NF)], axis=0)
    offs = (jnp.arange(NF, dtype=jnp.int32) * (VOCAB + 1))[None, :]
    idx_emb = (feat_ids.astype(jnp.int32) + offs).reshape(-1)
    B_emb = 40960
    idx_emb = jnp.concatenate(
        [idx_emb, jnp.zeros((B_emb - N * NF,), jnp.int32)])
    emb_rows = _gather_rows(table, idx_emb, C=128)
    e = emb_rows[: N * NF].reshape(N, NF * D)

    h = _proj_call(
        e,
        p["proj_W1"].T, p["proj_b1"].reshape(1, -1).astype(f32),
        p["proj_W2"].T, p["proj_b2"].reshape(1, -1).astype(f32),
        p["proj_W3"].T, p["proj_b3"].reshape(1, -1).astype(f32),
    )

    idx_f = _tmajor(src_forward)
    idx_b = _tmajor(src_backward)
    idx_m = _tmajor(src_repeat_next)

    bf16 = jnp.bfloat16
    for l in range(2):
        rf = _gather_rows(h, idx_f, C=200).reshape(DEG, N, D)
        rb = _gather_rows(h, idx_b, C=200).reshape(DEG, N, D)
        rm = _gather_rows(h, idx_m, C=200).reshape(DEG, N, D)
        pre = "l%d_" % l
        hf = _lstm_call(
            rf,
            p[pre + "forward_Wih"].T.astype(bf16),
            p[pre + "forward_Whh"].T.astype(bf16),
            (p[pre + "forward_bih"] + p[pre + "forward_bhh"]
             ).reshape(1, -1).astype(f32))
        hb = _lstm_call(
            rb,
            p[pre + "backward_Wih"].T.astype(bf16),
            p[pre + "backward_Whh"].T.astype(bf16),
            (p[pre + "backward_bih"] + p[pre + "backward_bhh"]
             ).reshape(1, -1).astype(f32))
        wss = (p[pre + "forward_Wself"] + p[pre + "backward_Wself"]
               + p[pre + "repeat_next_Wself"]).T
        bss = (p[pre + "forward_bself"] + p[pre + "backward_bself"]
               + p[pre + "repeat_next_bself"]).reshape(1, -1).astype(f32)
        h = _combine_call(
            h, hf, hb, rm,
            p[pre + "forward_Wneigh"].T,
            p[pre + "backward_Wneigh"].T,
            p[pre + "repeat_next_Wneigh"].T,
            wss, bss,
            p["ln%d_g" % l].reshape(1, -1).astype(f32),
            p["ln%d_b" % l].reshape(1, -1).astype(f32))

    onehot = (graph_ids[:, None] == jnp.arange(G, dtype=graph_ids.dtype)
              [None, :]).astype(f32)
    return _pool_call(
        h, onehot,
        p["cls_W1"].T, p["cls_b1"].reshape(1, -1).astype(f32),
        p["cls_W2"].T, p["cls_b2"].reshape(1, -1).astype(f32),
        p["cls_W3"].T, p["cls_b3"].reshape(1, -1).astype(f32),
    )


# speedup vs baseline: 4.4251x; 1.0124x over previous
import functools

import jax
import jax.numpy as jnp
from jax import lax
from jax.experimental import pallas as pl
from jax.experimental.pallas import tpu as pltpu
from jax.experimental.pallas import tpu_sc as plsc

N = 10000
DEG = 32
D = 128
NF = 4
VOCAB = 1000
G = 16
NCLS = 33

_SC_NC = 2
_SC_NS = 16
_SC_NW = _SC_NC * _SC_NS


@functools.lru_cache(maxsize=None)
def _make_sc_gather(V, B, C, W=D):
    del V
    bpw = B // _SC_NW
    assert B % _SC_NW == 0 and bpw % C == 0 and C % 8 == 0
    nch = bpw // C
    assert nch % 2 == 0
    mesh = plsc.VectorSubcoreMesh(core_axis_name="c", subcore_axis_name="s")

    npairs = nch // 2

    @functools.partial(
        pl.kernel,
        mesh=mesh,
        out_type=jax.ShapeDtypeStruct((B, W), jnp.float32),
        scratch_types=[
            pltpu.VMEM((bpw,), jnp.int32),
            pltpu.VMEM((C, W), jnp.float32),
            pltpu.VMEM((C, W), jnp.float32),
            pltpu.VMEM((C, W), jnp.float32),
            pltpu.VMEM((C, W), jnp.float32),
            pltpu.SemaphoreType.DMA,
            pltpu.SemaphoreType.DMA,
            pltpu.SemaphoreType.DMA,
            pltpu.SemaphoreType.DMA,
            pltpu.SemaphoreType.DMA,
            pltpu.SemaphoreType.DMA,
            pltpu.SemaphoreType.DMA,
            pltpu.SemaphoreType.DMA,
        ],
    )
    def gather_kernel(table_hbm, idx_hbm, out_hbm, idx_v,
                      b0, b1, b2, b3, g0, g1, g2, g3, w0, w1, w2, w3):
        wid = lax.axis_index("s") * _SC_NC + lax.axis_index("c")
        base = wid * bpw
        pltpu.sync_copy(idx_hbm.at[pl.ds(base, bpw)], idx_v)
        bufs = (b0, b1, b2, b3)
        gsems = (g0, g1, g2, g3)
        wsems = (w0, w1, w2, w3)

        def wait_wb(s, off):
            pltpu.make_async_copy(
                bufs[s], out_hbm.at[pl.ds(base + off, C)], wsems[s]).wait()

        def pair(p, slot, wait_prev):
            hs = []
            for k in range(2):
                s = 2 * slot + k
                o = (2 * p + k) * C
                if wait_prev:
                    wait_wb(s, o - 4 * C)
                hs.append(pltpu.async_copy(
                    table_hbm.at[idx_v.at[pl.ds(o, C)]], bufs[s], gsems[s]))
            for k in range(2):
                s = 2 * slot + k
                o = (2 * p + k) * C
                hs[k].wait()
                pltpu.async_copy(bufs[s], out_hbm.at[pl.ds(base + o, C)],
                                 wsems[s])

        pair(0, 0, False)
        if npairs > 1:
            pair(1, 1, False)

            def body(q, carry):
                pair(2 * q + 2, 0, True)
                pair(2 * q + 3, 1, True)
                return carry

            lax.fori_loop(0, (npairs - 2) // 2, body, 0)
            if npairs % 2 == 1:
                pair(npairs - 1, 0, True)

        if npairs == 1:
            s0p, s1p = 0, None
        elif npairs % 2 == 1:
            s0p, s1p = npairs - 1, npairs - 2
        else:
            s0p, s1p = npairs - 2, npairs - 1
        for k in range(2):
            wait_wb(k, (2 * s0p + k) * C)
        if s1p is not None:
            for k in range(2):
                wait_wb(2 + k, (2 * s1p + k) * C)

    return gather_kernel


def _gather_rows(table, idx, C):
    return _make_sc_gather(table.shape[0], idx.shape[0], C,
                           table.shape[1])(table, idx)


def _proj_body(e_ref, w1_ref, b1_ref, w2_ref, b2_ref, w3_ref, b3_ref, o_ref):
    h = jnp.dot(e_ref[...], w1_ref[...], preferred_element_type=jnp.float32)
    h = jnp.maximum(h + b1_ref[...], 0.0)
    h = jnp.dot(h, w2_ref[...], preferred_element_type=jnp.float32)
    h = jnp.maximum(h + b2_ref[...], 0.0)
    h = jnp.dot(h, w3_ref[...], preferred_element_type=jnp.float32)
    o_ref[...] = h + b3_ref[...]


def _proj_call(e, w1t, b1, w2t, b2, w3t, b3):
    TN = 1000
    nt = N // TN
    const = lambda shape: pl.BlockSpec(shape, lambda i: (0, 0))
    return pl.pallas_call(
        _proj_body,
        grid=(nt,),
        in_specs=[
            pl.BlockSpec((TN, NF * D), lambda i: (i, 0)),
            const(w1t.shape), const(b1.shape),
            const(w2t.shape), const(b2.shape),
            const(w3t.shape), const(b3.shape),
        ],
        out_specs=pl.BlockSpec((TN, D), lambda i: (i, 0)),
        out_shape=jax.ShapeDtypeStruct((N, D), jnp.float32),
    )(e, w1t, b1, w2t, b2, w3t, b3)


def _lstm_scan(m_ref, a1, a2, b2, tn):
    def step(t, hc):
        h, c = hc
        gates = (
            jnp.dot(m_ref[t].astype(jnp.bfloat16), a1,
                    preferred_element_type=jnp.float32)
            + jnp.dot(h.astype(jnp.bfloat16), a2,
                      preferred_element_type=jnp.float32)
            + b2
        )
        i = jax.nn.sigmoid(gates[:, 0 * D:1 * D])
        f = jax.nn.sigmoid(gates[:, 1 * D:2 * D])
        g = jnp.tanh(gates[:, 2 * D:3 * D])
        o = jax.nn.sigmoid(gates[:, 3 * D:4 * D])
        c = f * c + i * g
        h = o * jnp.tanh(c)
        return (h, c)

    z = jnp.zeros((tn, D), jnp.float32)
    h, _ = lax.fori_loop(0, DEG, step, (z, z))
    return h


def _make_lstm_body(tn):
    def body(m_ref, a1_ref, a2_ref, b2_ref, o_ref):
        o_ref[...] = _lstm_scan(m_ref, a1_ref[...], a2_ref[...], b2_ref[...],
                                tn)

    return body


def _lstm_call(m, a1, a2, b2, TN=1000):
    nt = N // TN
    const = lambda arr: pl.BlockSpec(arr.shape, lambda i: (0, 0))
    return pl.pallas_call(
        _make_lstm_body(TN),
        grid=(nt,),
        in_specs=[
            pl.BlockSpec((DEG, TN, D), lambda i: (0, i, 0)),
            const(a1), const(a2), const(b2),
        ],
        out_specs=pl.BlockSpec((TN, D), lambda i: (i, 0)),
        out_shape=jax.ShapeDtypeStruct((N, D), jnp.float32),
    )(m, a1, a2, b2)


def _combine_body(h_ref, hf_ref, hb_ref, mm_ref,
                  wnf_ref, wnb_ref, wnm_ref, wss_ref, bss_ref,
                  g_ref, bln_ref, o_ref):
    x = h_ref[...]
    hm = jnp.mean(mm_ref[...].astype(jnp.float32), axis=0)
    out = (
        jnp.dot(x, wss_ref[...], preferred_element_type=jnp.float32)
        + bss_ref[...]
        + jnp.dot(hf_ref[...], wnf_ref[...],
                  preferred_element_type=jnp.float32)
        + jnp.dot(hb_ref[...], wnb_ref[...],
                  preferred_element_type=jnp.float32)
        + jnp.dot(hm, wnm_ref[...], preferred_element_type=jnp.float32)
    )
    out = out * (1.0 / 3.0) + x
    mu = jnp.mean(out, axis=1, keepdims=True)
    var = jnp.mean((out - mu) ** 2, axis=1, keepdims=True)
    out = (out - mu) * jax.lax.rsqrt(var + 1e-5) * g_ref[...] + bln_ref[...]
    o_ref[...] = jnp.maximum(out, 0.0)


def _combine_call(h, hf, hb, mm, wnf, wnb, wnm, wss, bss, g, bln, TN=1000):
    nt = N // TN
    const = lambda arr: pl.BlockSpec(arr.shape, lambda i: (0, 0))
    nspec = pl.BlockSpec((TN, D), lambda i: (i, 0))
    return pl.pallas_call(
        _combine_body,
        grid=(nt,),
        in_specs=[
            nspec, nspec, nspec,
            pl.BlockSpec((DEG, TN, D), lambda i: (0, i, 0)),
            const(wnf), const(wnb), const(wnm), const(wss), const(bss),
            const(g), const(bln),
        ],
        out_specs=nspec,
        out_shape=jax.ShapeDtypeStruct((N, D), jnp.float32),
    )(h, hf, hb, mm, wnf, wnb, wnm, wss, bss, g, bln)


def _make_pool_body(nt):
    def body(h_ref, oh_ref, w1_ref, b1_ref, w2_ref, b2_ref, w3_ref, b3_ref,
             o_ref, acc_ref):
        i = pl.program_id(0)

        @pl.when(i == 0)
        def _init():
            acc_ref[...] = jnp.full((G, D), -jnp.inf, jnp.float32)

        h = h_ref[...]
        oh = oh_ref[...]
        for gidx in range(G):
            m = oh[:, gidx:gidx + 1] > 0.5
            vals = jnp.where(m, h, -jnp.inf)
            acc_ref[pl.ds(gidx, 1), :] = jnp.maximum(
                acc_ref[pl.ds(gidx, 1), :],
                jnp.max(vals, axis=0, keepdims=True))

        @pl.when(i == nt - 1)
        def _head():
            z = jnp.dot(acc_ref[...], w1_ref[...],
                        preferred_element_type=jnp.float32)
            z = jnp.maximum(z + b1_ref[...], 0.0)
            z = jnp.dot(z, w2_ref[...], preferred_element_type=jnp.float32)
            z = jnp.maximum(z + b2_ref[...], 0.0)
            z = jnp.dot(z, w3_ref[...], preferred_element_type=jnp.float32)
            o_ref[...] = z + b3_ref[...]

    return body


def _pool_call(h, onehot, w1t, b1, w2t, b2, w3t, b3):
    TN = 1000
    nt = N // TN
    const = lambda arr: pl.BlockSpec(arr.shape, lambda i: (0, 0))
    return pl.pallas_call(
        _make_pool_body(nt),
        grid=(nt,),
        in_specs=[
            pl.BlockSpec((TN, D), lambda i: (i, 0)),
            pl.BlockSpec((TN, G), lambda i: (i, 0)),
            const(w1t), const(b1), const(w2t), const(b2), const(w3t),
            const(b3),
        ],
        out_specs=pl.BlockSpec((G, NCLS), lambda i: (0, 0)),
        out_shape=jax.ShapeDtypeStruct((G, NCLS), jnp.float32),
        scratch_shapes=[pltpu.VMEM((G, D), jnp.float32)],
    )(h, onehot, w1t, b1, w2t, b2, w3t, b3)


def _tmajor(src):
    return src.reshape(N, DEG).T.reshape(-1).astype(jnp.int32)


def kernel(params, feat_ids, src_forward, src_backward, src_repeat_next,
           graph_ids):
    p = params
    f32 = jnp.float32

    table = jnp.concatenate([p["emb_%d" % i] for i in range(NF)], axis=0)
    offs = (jnp.arange(NF, dtype=jnp.int32) * (VOCAB + 1))[None, :]
    idx_emb = (feat_ids.astype(jnp.int32) + offs).reshape(-1)
    B_emb = 40960
    idx_emb = jnp.concatenate(
        [idx_emb, jnp.zeros((B_emb - N * NF,), jnp.int32)])
    emb_rows = _gather_rows(table, idx_emb, C=128)
    e = emb_rows[: N * NF].reshape(N, NF * D)

    h = _proj_call(
        e,
        p["proj_W1"].T, p["proj_b1"].reshape(1, -1).astype(f32),
        p["proj_W2"].T, p["proj_b2"].reshape(1, -1).astype(f32),
        p["proj_W3"].T, p["proj_b3"].reshape(1, -1).astype(f32),
    )

    idx_f = _tmajor(src_forward)
    idx_b = _tmajor(src_backward)
    idx_m = _tmajor(src_repeat_next)

    bf16 = jnp.bfloat16
    for l in range(2):
        rf = _gather_rows(h, idx_f, C=200).reshape(DEG, N, D)
        rb = _gather_rows(h, idx_b, C=200).reshape(DEG, N, D)
        rm = _gather_rows(h, idx_m, C=200).reshape(DEG, N, D)
        pre = "l%d_" % l
        hf = _lstm_call(
            rf,
            p[pre + "forward_Wih"].T.astype(bf16),
            p[pre + "forward_Whh"].T.astype(bf16),
            (p[pre + "forward_bih"] + p[pre + "forward_bhh"]
             ).reshape(1, -1).astype(f32))
        hb = _lstm_call(
            rb,
            p[pre + "backward_Wih"].T.astype(bf16),
            p[pre + "backward_Whh"].T.astype(bf16),
            (p[pre + "backward_bih"] + p[pre + "backward_bhh"]
             ).reshape(1, -1).astype(f32))
        wss = (p[pre + "forward_Wself"] + p[pre + "backward_Wself"]
               + p[pre + "repeat_next_Wself"]).T
        bss = (p[pre + "forward_bself"] + p[pre + "backward_bself"]
               + p[pre + "repeat_next_bself"]).reshape(1, -1).astype(f32)
        h = _combine_call(
            h, hf, hb, rm,
            p[pre + "forward_Wneigh"].T,
            p[pre + "backward_Wneigh"].T,
            p[pre + "repeat_next_Wneigh"].T,
            wss, bss,
            p["ln%d_g" % l].reshape(1, -1).astype(f32),
            p["ln%d_b" % l].reshape(1, -1).astype(f32))

    onehot = (graph_ids[:, None] == jnp.arange(G, dtype=graph_ids.dtype)
              [None, :]).astype(f32)
    return _pool_call(
        h, onehot,
        p["cls_W1"].T, p["cls_b1"].reshape(1, -1).astype(f32),
        p["cls_W2"].T, p["cls_b2"].reshape(1, -1).astype(f32),
        p["cls_W3"].T, p["cls_b3"].reshape(1, -1).astype(f32),
    )
